# baseline (speedup 1.0000x reference)
import functools

import jax
import jax.numpy as jnp
from jax import lax
from jax.experimental import pallas as pl
from jax.experimental.pallas import tpu as pltpu
from jax.experimental.pallas import tpu_sc as plsc

VOCAB = 1000000
DIM = 64
B = 4096
L = 200

NC = 2
NS = 16
NW = NC * NS
TOK = B * L
PER_W = TOK // NW
CHUNK = 128
NCH = PER_W // CHUNK
NBUF = 2


@functools.partial(
    pl.kernel,
    out_type=jax.ShapeDtypeStruct((TOK, DIM), jnp.float32),
    mesh=plsc.VectorSubcoreMesh(core_axis_name="c", subcore_axis_name="s"),
    compiler_params=pltpu.CompilerParams(use_tc_tiling_on_sc=False),
    scratch_types=[
        pltpu.VMEM((NCH, CHUNK), jnp.int32),
        pltpu.VMEM((NBUF, CHUNK, DIM), jnp.float32),
        pltpu.SemaphoreType.DMA,
        pltpu.SemaphoreType.DMA,
        pltpu.SemaphoreType.DMA,
        pltpu.SemaphoreType.DMA,
    ],
)
def _emb_lookup(idx_hbm, table_hbm, out_hbm, idx_v, rows_v, g0, g1, w0, w1):
    gsem = [g0, g1]
    wsem = [w0, w1]
    wid = lax.axis_index("s") * NC + lax.axis_index("c")
    base = wid * PER_W

    pltpu.sync_copy(idx_hbm.at[wid], idx_v)

    def gather_copy(j, b):
        return pltpu.make_async_copy(
            table_hbm.at[idx_v.at[j]], rows_v.at[b], gsem[b])

    def write_copy(j, b):
        return pltpu.make_async_copy(
            rows_v.at[b], out_hbm.at[pl.ds(base + j * CHUNK, CHUNK)], wsem[b])

    for b in range(NBUF):
        gather_copy(b, b).start()

    def outer(jj, _):
        for b in range(NBUF):
            j = jj * NBUF + b
            gather_copy(j, b).wait()
            write_copy(j, b).start()
            write_copy(j, b).wait()

            @pl.when(j + NBUF < NCH)
            def _():
                gather_copy(j + NBUF, b).start()
        return 0

    lax.fori_loop(0, NCH // NBUF, outer, 0)


def kernel(token_ids, table):
    idx = token_ids.reshape(NW, NCH, CHUNK)
    out = _emb_lookup(idx, table)
    return out.reshape(B, L, DIM)


# speedup vs baseline: 1.0168x; 1.0168x over previous
import functools

import jax
import jax.numpy as jnp
from jax import lax
from jax.experimental import pallas as pl
from jax.experimental.pallas import tpu as pltpu
from jax.experimental.pallas import tpu_sc as plsc

VOCAB = 1000000
DIM = 64
B = 4096
L = 200

NC = 2
NS = 16
NW = NC * NS
TOK = B * L
PER_W = TOK // NW
CHUNK = 128
NCH = PER_W // CHUNK
NBUF = 4
LOOK = 2


@functools.partial(
    pl.kernel,
    out_type=jax.ShapeDtypeStruct((TOK, DIM), jnp.float32),
    mesh=plsc.VectorSubcoreMesh(core_axis_name="c", subcore_axis_name="s"),
    compiler_params=pltpu.CompilerParams(use_tc_tiling_on_sc=False),
    scratch_types=[
        pltpu.VMEM((NCH, CHUNK), jnp.int32),
        pltpu.VMEM((NBUF, CHUNK, DIM), jnp.float32),
    ] + [pltpu.SemaphoreType.DMA] * (2 * NBUF),
)
def _emb_lookup(idx_hbm, table_hbm, out_hbm, idx_v, rows_v, *sems):
    gsem = sems[:NBUF]
    wsem = sems[NBUF:]
    wid = lax.axis_index("s") * NC + lax.axis_index("c")
    base = wid * PER_W

    pltpu.sync_copy(idx_hbm.at[wid], idx_v)

    def gather_copy(j, b):
        return pltpu.make_async_copy(
            table_hbm.at[idx_v.at[j]], rows_v.at[b], gsem[b])

    def write_copy(j, b):
        return pltpu.make_async_copy(
            rows_v.at[b], out_hbm.at[pl.ds(base + j * CHUNK, CHUNK)], wsem[b])

    for t in range(LOOK):
        gather_copy(t, t % NBUF).start()

    for t in range(LOOK):
        b, b2 = t % NBUF, (t + LOOK) % NBUF
        gather_copy(t, b).wait()
        write_copy(t, b).start()
        gather_copy(t + LOOK, b2).start()

    n_steady = NCH - 2 * LOOK

    def outer(jj, _):
        for i in range(NBUF):
            t = LOOK + jj * NBUF + i
            b, b2 = (LOOK + i) % NBUF, i % NBUF
            gather_copy(t, b).wait()
            write_copy(t, b).start()
            write_copy(t - LOOK, b2).wait()
            gather_copy(t + LOOK, b2).start()
        return 0

    lax.fori_loop(0, n_steady // NBUF, outer, 0)

    for t in range(NCH - LOOK, NCH):
        b, b2 = t % NBUF, (t + LOOK) % NBUF
        gather_copy(t, b).wait()
        write_copy(t, b).start()
        write_copy(t - LOOK, b2).wait()

    for t in range(NCH - LOOK, NCH):
        write_copy(t, t % NBUF).wait()


def kernel(token_ids, table):
    idx = token_ids.reshape(NW, NCH, CHUNK)
    out = _emb_lookup(idx, table)
    return out.reshape(B, L, DIM)
